# baseline (speedup 1.0000x reference)
import jax
import jax.numpy as jnp
from jax import lax
from jax.experimental import pallas as pl
from jax.experimental.pallas import tpu as pltpu

_NUM_CLASSES = 1000
_PAD_CLASSES = 1024
_TEMPERATURE = 0.05
_EPS = 1e-12


_LOG2E = 1.4426950408889634
_LN2 = 0.6931471805599453
_BATCH_BLOCK = 2048


def _loss_block(embT_ref, wT_ref, lab_ref, out_ref, cl_ref):
    wt = wT_ref[...]
    d = wt.shape[0]
    wn = wt * (
        (_LOG2E / _TEMPERATURE)
        / (jnp.sqrt(jnp.sum(wt * wt, axis=0, keepdims=True)) + _EPS)
    )
    wn65 = jnp.concatenate([wn, jnp.zeros((1, _NUM_CLASSES), jnp.float32)], axis=0)
    pad_cols = jnp.concatenate(
        [
            jnp.zeros((d, _PAD_CLASSES - _NUM_CLASSES), jnp.float32),
            jnp.full((1, _PAD_CLASSES - _NUM_CLASSES), -1e30, jnp.float32),
        ],
        axis=0,
    )
    wbf = jnp.concatenate([wn65, pad_cols], axis=1).astype(jnp.bfloat16)

    et = embT_ref[...]
    en = et / (jnp.sqrt(jnp.sum(et * et, axis=0, keepdims=True)) + _EPS)
    en65 = jnp.concatenate([en, jnp.ones((1, et.shape[1]), jnp.float32)], axis=0)

    lgT = lax.dot_general(
        wbf,
        en65.astype(jnp.bfloat16),
        (((0,), (0,)), ((), ())),
        preferred_element_type=jnp.float32,
    )

    lab = lab_ref[0, 0, :]
    row = lax.broadcasted_iota(jnp.int32, lgT.shape, 0)
    ex = jnp.exp2(lgT)
    masked = jnp.where(row == lab[None, :], lgT, 0.0)
    ones_row = jnp.ones((1, lgT.shape[0]), jnp.float32)
    s = lax.dot_general(
        ones_row, ex, (((1,), (0,)), ((), ())), preferred_element_type=jnp.float32
    )[0]
    picked2 = lax.dot_general(
        ones_row, masked, (((1,), (0,)), ((), ())), preferred_element_type=jnp.float32
    )[0]
    lse = jnp.log(s)

    loss = jnp.sum(lse - picked2 * _LN2) * (1.0 / lgT.shape[1])
    out_ref[...] = loss.reshape(1, 1)

    b = lab.shape[0]
    cl_ref[0, pl.ds(0, b)] = lab
    cl_ref[0, pl.ds(b, b)] = lab


def _fused_tc(embeddings, labels_i32, W):
    b, d = embeddings.shape
    labs3 = labels_i32.reshape(1, 1, b)
    loss, cl = pl.pallas_call(
        _loss_block,
        in_specs=[
            pl.BlockSpec((d, b), lambda: (0, 0)),
            pl.BlockSpec((d, _NUM_CLASSES), lambda: (0, 0)),
            pl.BlockSpec((1, 1, b), lambda: (0, 0, 0)),
        ],
        out_specs=[
            pl.BlockSpec((1, 1), lambda: (0, 0)),
            pl.BlockSpec((1, 2 * b), lambda: (0, 0)),
        ],
        out_shape=[
            jax.ShapeDtypeStruct((1, 1), jnp.float32),
            jax.ShapeDtypeStruct((1, 2 * b), jnp.int32),
        ],
    )(embeddings.T, W.T, labs3)
    return loss[0, 0], cl.reshape(2 * b)


def kernel(embeddings, labels, W):
    labels_i32 = labels.astype(jnp.int32)
    loss, combined_labels = _fused_tc(embeddings, labels_i32, W)
    return (loss, combined_labels.astype(labels.dtype))


# speedup vs baseline: 1.2207x; 1.2207x over previous
import jax
import jax.numpy as jnp
from jax import lax
from jax.experimental import pallas as pl
from jax.experimental.pallas import tpu as pltpu

_NUM_CLASSES = 1000
_PAD_CLASSES = 1024
_TEMPERATURE = 0.05
_EPS = 1e-12


_LOG2E = 1.4426950408889634
_LN2 = 0.6931471805599453
_BATCH_BLOCK = 2048


def _loss_block(embT_ref, wT_ref, lab_ref, out_ref, cl_ref):
    wt = wT_ref[...]
    d = wt.shape[0]
    wn = wt * (
        (_LOG2E / _TEMPERATURE)
        / (jnp.sqrt(jnp.sum(wt * wt, axis=0, keepdims=True)) + _EPS)
    )
    wn65 = jnp.concatenate([wn, jnp.zeros((1, _NUM_CLASSES), jnp.float32)], axis=0)
    pad_cols = jnp.concatenate(
        [
            jnp.zeros((d, _PAD_CLASSES - _NUM_CLASSES), jnp.float32),
            jnp.full((1, _PAD_CLASSES - _NUM_CLASSES), -1e30, jnp.float32),
        ],
        axis=0,
    )
    wbf = jnp.concatenate([wn65, pad_cols], axis=1).astype(jnp.bfloat16)

    et = embT_ref[...]
    en = et / (jnp.sqrt(jnp.sum(et * et, axis=0, keepdims=True)) + _EPS)
    en65 = jnp.concatenate([en, jnp.ones((1, et.shape[1]), jnp.float32)], axis=0)

    lgT = lax.dot_general(
        wbf,
        en65.astype(jnp.bfloat16),
        (((0,), (0,)), ((), ())),
        preferred_element_type=jnp.float32,
    )

    s = jnp.sum(jnp.exp2(lgT), axis=0)
    lse = jnp.log(s)

    lab = lab_ref[0, 0, :]
    row = lax.broadcasted_iota(jnp.int32, lgT.shape, 0)
    picked2 = jnp.sum(jnp.where(row == lab[None, :], lgT, 0.0), axis=0)

    loss = jnp.sum(lse - picked2 * _LN2) * (1.0 / lgT.shape[1])
    out_ref[...] = loss.reshape(1, 1)

    b = lab.shape[0]
    cl_ref[0, pl.ds(0, b)] = lab
    cl_ref[0, pl.ds(b, b)] = lab


def _fused_tc(embeddings, labels_i32, W):
    b, d = embeddings.shape
    labs3 = labels_i32.reshape(1, 1, b)
    loss, cl = pl.pallas_call(
        _loss_block,
        in_specs=[
            pl.BlockSpec((d, b), lambda: (0, 0)),
            pl.BlockSpec((d, _NUM_CLASSES), lambda: (0, 0)),
            pl.BlockSpec((1, 1, b), lambda: (0, 0, 0)),
        ],
        out_specs=[
            pl.BlockSpec((1, 1), lambda: (0, 0)),
            pl.BlockSpec((1, 2 * b), lambda: (0, 0)),
        ],
        out_shape=[
            jax.ShapeDtypeStruct((1, 1), jnp.float32),
            jax.ShapeDtypeStruct((1, 2 * b), jnp.int32),
        ],
    )(embeddings.T, W.T, labs3)
    return loss[0, 0], cl.reshape(2 * b)


def kernel(embeddings, labels, W):
    labels_i32 = labels.astype(jnp.int32)
    loss, combined_labels = _fused_tc(embeddings, labels_i32, W)
    return (loss, combined_labels.astype(labels.dtype))
